# baseline (speedup 1.0000x reference)
import functools

import jax
import jax.numpy as jnp
from jax import lax
from jax.experimental import pallas as pl
from jax.experimental.pallas import tpu as pltpu
from jax.experimental.pallas import tpu_sc as plsc

F32 = jnp.float32
I32 = jnp.int32

NW = 32
SY = 42
SZ = 40 * 42 * 42
GRID_R = 94880
GRID_G = NW * GRID_R


def _iota16():
    return lax.iota(I32, 16)


def _mesh():
    return plsc.VectorSubcoreMesh(core_axis_name="c", subcore_axis_name="s")


_SC_PARAMS = pltpu.CompilerParams(
    needs_layout_passes=False, use_tc_tiling_on_sc=False)


def _wid():
    return lax.axis_index("s") * 2 + lax.axis_index("c")


def _build_lin_kernel(npad):
    P = npad // NW
    C = 1600
    nch = P // C

    @functools.partial(
        pl.kernel,
        mesh=_mesh(),
        compiler_params=_SC_PARAMS,
        out_type=jax.ShapeDtypeStruct((npad,), I32),
        scratch_types=[
            pltpu.VMEM((C * 4,), I32),
            pltpu.VMEM((C,), I32),
        ],
    )
    def lin_kernel(coords_hbm, lin_hbm, cbuf, lbuf):
        w = _wid()

        def chunk(ci, _):
            start = w * P + ci * C
            pltpu.sync_copy(coords_hbm.at[pl.ds(start * 4, C * 4)], cbuf)

            def grp(t, _):
                base = t * 64
                i4 = _iota16() * 4
                b = plsc.load_gather(cbuf, [i4 + base])
                z = plsc.load_gather(cbuf, [i4 + (base + 1)])
                y = plsc.load_gather(cbuf, [i4 + (base + 2)])
                x = plsc.load_gather(cbuf, [i4 + (base + 3)])
                linv = ((z + 1) * 40 + b) * (SY * SY) + (y + 1) * SY + (x + 1)
                lbuf[pl.ds(t * 16, 16)] = linv
                return 0

            lax.fori_loop(0, C // 16, grp, 0)
            pltpu.sync_copy(lbuf, lin_hbm.at[pl.ds(start, C)])
            return 0

        lax.fori_loop(0, nch, chunk, 0)

    return lin_kernel


def _build_grid_kernel(n):
    C = 2000
    nch = n // C

    @functools.partial(
        pl.kernel,
        mesh=_mesh(),
        compiler_params=_SC_PARAMS,
        out_type=jax.ShapeDtypeStruct((GRID_G,), I32),
        scratch_types=[
            pltpu.VMEM((GRID_R,), I32),
            pltpu.VMEM((C,), I32),
            pltpu.VMEM((C,), I32),
        ],
    )
    def grid_kernel(lin_hbm, val_hbm, grid_hbm, local, lbuf, vbuf):
        w = _wid()
        base = w * GRID_R

        def init(i, _):
            local[pl.ds(i * 16, 16)] = jnp.full((16,), n * 27, I32)
            return 0

        lax.fori_loop(0, GRID_R // 16, init, 0)

        def chunk(ci, _):
            pltpu.sync_copy(lin_hbm.at[pl.ds(ci * C, C)], lbuf)
            pltpu.sync_copy(val_hbm.at[pl.ds(ci * C, C)], vbuf)

            def grp(t, _):
                l = lbuf[pl.ds(t * 16, 16)]
                v = vbuf[pl.ds(t * 16, 16)]
                inr = (l >= base) & (l < base + GRID_R)
                il = jnp.clip(l - base, 0, GRID_R - 1)
                plsc.store_scatter(local, [il], v, mask=inr)
                return 0

            lax.fori_loop(0, C // 16, grp, 0)
            return 0

        lax.fori_loop(0, nch, chunk, 0)
        pltpu.sync_copy(local, grid_hbm.at[pl.ds(base, GRID_R)])

    return grid_kernel


def _build_acc_kernel(npad, n):
    P = npad // NW
    C = 1280
    nch = P // C
    ng = C // 16

    @functools.partial(
        pl.kernel,
        mesh=_mesh(),
        compiler_params=_SC_PARAMS,
        out_type=jax.ShapeDtypeStruct((npad, 16), F32),
        scratch_types=[
            pltpu.VMEM((C,), I32),
            pltpu.VMEM((27 * C,), I32),
            pltpu.VMEM((27 * C,), I32),
            pltpu.VMEM((C, 16), F32),
            pltpu.SemaphoreType.DMA,
        ],
    )
    def acc_kernel(lin_hbm, grid_hbm, y_hbm, out_hbm, lbuf, ibuf, nbuf, accbuf, sem):
        w = _wid()

        def chunk(ci, _):
            start = w * P + ci * C
            pltpu.sync_copy(lin_hbm.at[pl.ds(start, C)], lbuf)

            def zero(r, _):
                accbuf[r, :] = jnp.zeros((16,), F32)
                return 0

            lax.fori_loop(0, C, zero, 0)

            def ka(k, _):
                dz = k // 9 - 1
                dy = (k // 3) % 3 - 1
                dx = k % 3 - 1
                delta = dz * SZ + dy * SY + dx

                def grp(t, _):
                    ibuf[pl.ds(k * C + t * 16, 16)] = lbuf[pl.ds(t * 16, 16)] + delta
                    return 0

                lax.fori_loop(0, ng, grp, 0)
                return 0

            lax.fori_loop(0, 27, ka, 0)

            pltpu.async_copy(grid_hbm.at[ibuf], nbuf, sem).wait()

            def kc(k, _):
                def grp(t, _):
                    ibuf[pl.ds(k * C + t * 16, 16)] = jnp.minimum(
                        nbuf[pl.ds(k * C + t * 16, 16)] + k, n * 27)
                    return 0

                lax.fori_loop(0, ng, grp, 0)
                return 0

            lax.fori_loop(0, 27, kc, 0)

            def fire_y(k, _):
                pltpu.async_copy(
                    y_hbm.at[ibuf.at[pl.ds(k * C, C)]],
                    accbuf, sem, add=True)
                return 0

            lax.fori_loop(0, 27, fire_y, 0)

            def drain_y(j, _):
                pltpu.make_async_copy(
                    y_hbm.at[ibuf.at[pl.ds(0, C)]],
                    accbuf, sem).wait()
                return 0

            lax.fori_loop(0, 27, drain_y, 0)

            pltpu.sync_copy(accbuf, out_hbm.at[pl.ds(start, C), :])
            return 0

        lax.fori_loop(0, nch, chunk, 0)

    return acc_kernel


def _matmul(x, w, npad, kdim, blk):
    def body(xr, wr, yr):
        yr[...] = jnp.dot(xr[...], wr[...], preferred_element_type=F32)

    return pl.pallas_call(
        body,
        grid=(npad // blk,),
        in_specs=[pl.BlockSpec((blk, kdim), lambda i: (i, 0)),
                  pl.BlockSpec((kdim, 432), lambda i: (0, 0))],
        out_specs=pl.BlockSpec((blk, 432), lambda i: (i, 0)),
        out_shape=jax.ShapeDtypeStruct((npad, 432), F32),
    )(x, w)


def _stats(acc, npad, n, blk):
    nb = npad // blk

    def body(ar, sr):
        i = pl.program_id(0)

        @pl.when(i == 0)
        def _():
            sr[...] = jnp.zeros((8, 16), F32)

        x = ar[...]
        sr[0:1, :] = sr[0:1, :] + jnp.sum(x, axis=0, keepdims=True)
        sr[1:2, :] = sr[1:2, :] + jnp.sum(x * x, axis=0, keepdims=True)

        @pl.when(i == nb - 1)
        def _():
            mean = sr[0:1, :] * (1.0 / n)
            var = sr[1:2, :] * (1.0 / n) - mean * mean
            sr[0:1, :] = mean
            sr[1:2, :] = var

    return pl.pallas_call(
        body,
        grid=(nb,),
        in_specs=[pl.BlockSpec((blk, 16), lambda i: (i, 0))],
        out_specs=pl.BlockSpec((8, 16), lambda i: (0, 0)),
        out_shape=jax.ShapeDtypeStruct((8, 16), F32),
    )(acc)


def _bn_matmul(acc, stats, gb, w, npad, n, blk):
    def body(ar, sr, gr, wr, yr):
        i = pl.program_id(0)
        scale = gr[0:1, :] * lax.rsqrt(sr[1:2, :] + 1e-3)
        shift = gr[1:2, :] - sr[0:1, :] * scale
        x1 = jnp.maximum(ar[...] * scale + shift, 0.0)
        rows = lax.broadcasted_iota(I32, (blk, 1), 0) + i * blk
        x1 = jnp.where(rows < n, x1, 0.0)
        yr[...] = jnp.dot(x1, wr[...], preferred_element_type=F32)

    return pl.pallas_call(
        body,
        grid=(npad // blk,),
        in_specs=[pl.BlockSpec((blk, 16), lambda i: (i, 0)),
                  pl.BlockSpec((8, 16), lambda i: (0, 0)),
                  pl.BlockSpec((8, 16), lambda i: (0, 0)),
                  pl.BlockSpec((16, 432), lambda i: (0, 0))],
        out_specs=pl.BlockSpec((blk, 432), lambda i: (i, 0)),
        out_shape=jax.ShapeDtypeStruct((npad, 432), F32),
    )(acc, stats, gb, w)


def _bn_out(acc, stats, gb, npad, blk):
    def body(ar, sr, gr, yr):
        scale = gr[0:1, :] * lax.rsqrt(sr[1:2, :] + 1e-3)
        shift = gr[1:2, :] - sr[0:1, :] * scale
        yr[...] = jnp.maximum(ar[...] * scale + shift, 0.0)

    return pl.pallas_call(
        body,
        grid=(npad // blk,),
        in_specs=[pl.BlockSpec((blk, 16), lambda i: (i, 0)),
                  pl.BlockSpec((8, 16), lambda i: (0, 0)),
                  pl.BlockSpec((8, 16), lambda i: (0, 0))],
        out_specs=pl.BlockSpec((blk, 16), lambda i: (i, 0)),
        out_shape=jax.ShapeDtypeStruct((npad, 16), F32),
    )(acc, stats, gb)


def kernel(features, coords, W1, g1, b1, W2, g2, b2, batch_size=40):
    n = features.shape[0]
    unit = NW * 1280
    npad = ((n + unit - 1) // unit) * unit

    pad_row = jnp.array([[0, 41, 20, 20]], I32)
    cpad = jnp.concatenate(
        [coords.astype(I32), jnp.broadcast_to(pad_row, (npad - n, 4))], axis=0
    ).reshape(-1)
    xpad = jnp.zeros((npad, 3), F32).at[:n].set(features)
    w1c = jnp.transpose(W1, (1, 0, 2)).reshape(3, 27 * 16)
    w2c = jnp.transpose(W2, (1, 0, 2)).reshape(16, 27 * 16)
    gb1 = jnp.zeros((8, 16), F32).at[0].set(g1).at[1].set(b1)
    gb2 = jnp.zeros((8, 16), F32).at[0].set(g2).at[1].set(b2)

    cI = cpad.reshape(npad, 4)
    grid4 = jnp.full((40, 40, 100, 100), -1, I32).at[
        cI[:n, 0], cI[:n, 1], cI[:n, 2], cI[:n, 3]].set(jnp.arange(n, dtype=I32))
    win = grid4[cI[:n, 0], cI[:n, 1], cI[:n, 2], cI[:n, 3]] * 27

    lin = _build_lin_kernel(npad)(cpad)
    grid = _build_grid_kernel(n)(lin, win)

    y1 = _matmul(xpad, w1c, npad, 3, 2048).reshape(npad * 27, 16)
    acc1 = _build_acc_kernel(npad, n)(lin, grid, y1)
    st1 = _stats(acc1, npad, n, 4096)

    y2 = _bn_matmul(acc1, st1, gb1, w2c, npad, n, 2048).reshape(npad * 27, 16)
    acc2 = _build_acc_kernel(npad, n)(lin, grid, y2)
    st2 = _stats(acc2, npad, n, 4096)

    return _bn_out(acc2, st2, gb2, npad, 4096)[:n]


# speedup vs baseline: 8.8107x; 8.8107x over previous
import functools

import jax
import jax.numpy as jnp
from jax import lax
from jax.experimental import pallas as pl
from jax.experimental.pallas import tpu as pltpu
from jax.experimental.pallas import tpu_sc as plsc

F32 = jnp.float32
I32 = jnp.int32

NW = 32
SY = 42
SZ = 40 * 42 * 42
GRID_R = 94880
GRID_G = NW * GRID_R


def _iota16():
    return lax.iota(I32, 16)


def _mesh():
    return plsc.VectorSubcoreMesh(core_axis_name="c", subcore_axis_name="s")


_SC_PARAMS = pltpu.CompilerParams(
    needs_layout_passes=False, use_tc_tiling_on_sc=False)


def _wid():
    return lax.axis_index("s") * 2 + lax.axis_index("c")


def _build_lin_kernel(npad):
    P = npad // NW
    C = 1600
    nch = P // C

    @functools.partial(
        pl.kernel,
        mesh=_mesh(),
        compiler_params=_SC_PARAMS,
        out_type=jax.ShapeDtypeStruct((npad,), I32),
        scratch_types=[
            pltpu.VMEM((C * 4,), I32),
            pltpu.VMEM((C,), I32),
        ],
    )
    def lin_kernel(coords_hbm, lin_hbm, cbuf, lbuf):
        w = _wid()

        def chunk(ci, _):
            start = w * P + ci * C
            pltpu.sync_copy(coords_hbm.at[pl.ds(start * 4, C * 4)], cbuf)

            def grp(t, _):
                base = t * 64
                i4 = _iota16() * 4
                b = plsc.load_gather(cbuf, [i4 + base])
                z = plsc.load_gather(cbuf, [i4 + (base + 1)])
                y = plsc.load_gather(cbuf, [i4 + (base + 2)])
                x = plsc.load_gather(cbuf, [i4 + (base + 3)])
                linv = ((z + 1) * 40 + b) * (SY * SY) + (y + 1) * SY + (x + 1)
                lbuf[pl.ds(t * 16, 16)] = linv
                return 0

            lax.fori_loop(0, C // 16, grp, 0)
            pltpu.sync_copy(lbuf, lin_hbm.at[pl.ds(start, C)])
            return 0

        lax.fori_loop(0, nch, chunk, 0)

    return lin_kernel


def _build_grid_kernel(n):
    C = 2000
    nch = n // C

    @functools.partial(
        pl.kernel,
        mesh=_mesh(),
        compiler_params=_SC_PARAMS,
        out_type=jax.ShapeDtypeStruct((GRID_G,), I32),
        scratch_types=[
            pltpu.VMEM((GRID_R,), I32),
            pltpu.VMEM((C,), I32),
            pltpu.VMEM((C,), I32),
        ],
    )
    def grid_kernel(lin_hbm, val_hbm, grid_hbm, local, lbuf, vbuf):
        w = _wid()
        base = w * GRID_R

        def init(i, _):
            local[pl.ds(i * 16, 16)] = jnp.full((16,), n * 27, I32)
            return 0

        lax.fori_loop(0, GRID_R // 16, init, 0)

        def chunk(ci, _):
            pltpu.sync_copy(lin_hbm.at[pl.ds(ci * C, C)], lbuf)
            pltpu.sync_copy(val_hbm.at[pl.ds(ci * C, C)], vbuf)

            def grp(t, _):
                l = lbuf[pl.ds(t * 16, 16)]
                v = vbuf[pl.ds(t * 16, 16)]
                inr = (l >= base) & (l < base + GRID_R)
                il = jnp.clip(l - base, 0, GRID_R - 1)
                plsc.store_scatter(local, [il], v, mask=inr)
                return 0

            lax.fori_loop(0, C // 16, grp, 0)
            return 0

        lax.fori_loop(0, nch, chunk, 0)
        pltpu.sync_copy(local, grid_hbm.at[pl.ds(base, GRID_R)])

    return grid_kernel


def _build_acc_kernel(npad, n):
    P = npad // NW
    C = 1280
    nch = P // C
    ng = C // 16

    @functools.partial(
        pl.kernel,
        mesh=_mesh(),
        compiler_params=_SC_PARAMS,
        out_type=jax.ShapeDtypeStruct((npad, 16), F32),
        scratch_types=[
            pltpu.VMEM((C,), I32),
            pltpu.VMEM((27 * C,), I32),
            pltpu.VMEM((27 * C,), I32),
            pltpu.VMEM((C, 16), F32),
            pltpu.SemaphoreType.DMA,
        ],
    )
    def acc_kernel(lin_hbm, grid_hbm, y_hbm, out_hbm, lbuf, ibuf, nbuf, accbuf, sem):
        w = _wid()

        def chunk(ci, _):
            start = w * P + ci * C
            pltpu.sync_copy(lin_hbm.at[pl.ds(start, C)], lbuf)

            def zero(r, _):
                accbuf[r, :] = jnp.zeros((16,), F32)
                return 0

            lax.fori_loop(0, C, zero, 0)

            def ka(k, _):
                dz = k // 9 - 1
                dy = (k // 3) % 3 - 1
                dx = k % 3 - 1
                delta = dz * SZ + dy * SY + dx

                def grp(t, _):
                    ibuf[pl.ds(k * C + t * 16, 16)] = lbuf[pl.ds(t * 16, 16)] + delta
                    return 0

                lax.fori_loop(0, ng, grp, 0)
                return 0

            lax.fori_loop(0, 27, ka, 0)

            pltpu.async_copy(grid_hbm.at[ibuf], nbuf, sem).wait()

            def kc(k, _):
                def grp(t, _):
                    nb = nbuf[pl.ds(k * C + t * 16, 16)]
                    zr = (n + t * 16 + _iota16()) * 27 + k
                    ibuf[pl.ds(k * C + t * 16, 16)] = jnp.where(
                        nb < n * 27, nb + k, zr)
                    return 0

                lax.fori_loop(0, ng, grp, 0)
                return 0

            lax.fori_loop(0, 27, kc, 0)

            def fire_y(k, _):
                pltpu.async_copy(
                    y_hbm.at[ibuf.at[pl.ds(k * C, C)]],
                    accbuf, sem, add=True)
                return 0

            lax.fori_loop(0, 27, fire_y, 0)

            def drain_y(j, _):
                pltpu.make_async_copy(
                    y_hbm.at[ibuf.at[pl.ds(0, C)]],
                    accbuf, sem).wait()
                return 0

            lax.fori_loop(0, 27, drain_y, 0)

            pltpu.sync_copy(accbuf, out_hbm.at[pl.ds(start, C), :])
            return 0

        lax.fori_loop(0, nch, chunk, 0)

    return acc_kernel


def _matmul(x, w, npad, kdim, blk):
    def body(xr, wr, yr):
        yr[...] = jnp.dot(xr[...], wr[...], preferred_element_type=F32)

    return pl.pallas_call(
        body,
        grid=(npad // blk,),
        in_specs=[pl.BlockSpec((blk, kdim), lambda i: (i, 0)),
                  pl.BlockSpec((kdim, 432), lambda i: (0, 0))],
        out_specs=pl.BlockSpec((blk, 432), lambda i: (i, 0)),
        out_shape=jax.ShapeDtypeStruct((npad, 432), F32),
    )(x, w)


def _stats(acc, npad, n, blk):
    nb = npad // blk

    def body(ar, sr):
        i = pl.program_id(0)

        @pl.when(i == 0)
        def _():
            sr[...] = jnp.zeros((8, 16), F32)

        x = ar[...]
        sr[0:1, :] = sr[0:1, :] + jnp.sum(x, axis=0, keepdims=True)
        sr[1:2, :] = sr[1:2, :] + jnp.sum(x * x, axis=0, keepdims=True)

        @pl.when(i == nb - 1)
        def _():
            mean = sr[0:1, :] * (1.0 / n)
            var = sr[1:2, :] * (1.0 / n) - mean * mean
            sr[0:1, :] = mean
            sr[1:2, :] = var

    return pl.pallas_call(
        body,
        grid=(nb,),
        in_specs=[pl.BlockSpec((blk, 16), lambda i: (i, 0))],
        out_specs=pl.BlockSpec((8, 16), lambda i: (0, 0)),
        out_shape=jax.ShapeDtypeStruct((8, 16), F32),
    )(acc)


def _bn_matmul(acc, stats, gb, w, npad, n, blk):
    def body(ar, sr, gr, wr, yr):
        i = pl.program_id(0)
        scale = gr[0:1, :] * lax.rsqrt(sr[1:2, :] + 1e-3)
        shift = gr[1:2, :] - sr[0:1, :] * scale
        x1 = jnp.maximum(ar[...] * scale + shift, 0.0)
        rows = lax.broadcasted_iota(I32, (blk, 1), 0) + i * blk
        x1 = jnp.where(rows < n, x1, 0.0)
        yr[...] = jnp.dot(x1, wr[...], preferred_element_type=F32)

    return pl.pallas_call(
        body,
        grid=(npad // blk,),
        in_specs=[pl.BlockSpec((blk, 16), lambda i: (i, 0)),
                  pl.BlockSpec((8, 16), lambda i: (0, 0)),
                  pl.BlockSpec((8, 16), lambda i: (0, 0)),
                  pl.BlockSpec((16, 432), lambda i: (0, 0))],
        out_specs=pl.BlockSpec((blk, 432), lambda i: (i, 0)),
        out_shape=jax.ShapeDtypeStruct((npad, 432), F32),
    )(acc, stats, gb, w)


def _bn_out(acc, stats, gb, npad, blk):
    def body(ar, sr, gr, yr):
        scale = gr[0:1, :] * lax.rsqrt(sr[1:2, :] + 1e-3)
        shift = gr[1:2, :] - sr[0:1, :] * scale
        yr[...] = jnp.maximum(ar[...] * scale + shift, 0.0)

    return pl.pallas_call(
        body,
        grid=(npad // blk,),
        in_specs=[pl.BlockSpec((blk, 16), lambda i: (i, 0)),
                  pl.BlockSpec((8, 16), lambda i: (0, 0)),
                  pl.BlockSpec((8, 16), lambda i: (0, 0))],
        out_specs=pl.BlockSpec((blk, 16), lambda i: (i, 0)),
        out_shape=jax.ShapeDtypeStruct((npad, 16), F32),
    )(acc, stats, gb)


def kernel(features, coords, W1, g1, b1, W2, g2, b2, batch_size=40):
    n = features.shape[0]
    unit = NW * 1280
    npad = ((n + unit - 1) // unit) * unit

    pad_row = jnp.array([[0, 41, 20, 20]], I32)
    cpad = jnp.concatenate(
        [coords.astype(I32), jnp.broadcast_to(pad_row, (npad - n, 4))], axis=0
    ).reshape(-1)
    xpad = jnp.zeros((npad, 3), F32).at[:n].set(features)
    w1c = jnp.transpose(W1, (1, 0, 2)).reshape(3, 27 * 16)
    w2c = jnp.transpose(W2, (1, 0, 2)).reshape(16, 27 * 16)
    gb1 = jnp.zeros((8, 16), F32).at[0].set(g1).at[1].set(b1)
    gb2 = jnp.zeros((8, 16), F32).at[0].set(g2).at[1].set(b2)

    cI = cpad.reshape(npad, 4)
    grid4 = jnp.full((40, 40, 100, 100), -1, I32).at[
        cI[:n, 0], cI[:n, 1], cI[:n, 2], cI[:n, 3]].set(jnp.arange(n, dtype=I32))
    win = grid4[cI[:n, 0], cI[:n, 1], cI[:n, 2], cI[:n, 3]] * 27

    lin = _build_lin_kernel(npad)(cpad)
    grid = _build_grid_kernel(n)(lin, win)

    y1 = _matmul(xpad, w1c, npad, 3, 2048).reshape(npad * 27, 16)
    acc1 = _build_acc_kernel(npad, n)(lin, grid, y1)
    st1 = _stats(acc1, npad, n, 4096)

    y2 = _bn_matmul(acc1, st1, gb1, w2c, npad, n, 2048).reshape(npad * 27, 16)
    acc2 = _build_acc_kernel(npad, n)(lin, grid, y2)
    st2 = _stats(acc2, npad, n, 4096)

    return _bn_out(acc2, st2, gb2, npad, 4096)[:n]
